# baseline (speedup 1.0000x reference)
import jax
import jax.numpy as jnp
from jax import lax
from jax.experimental import pallas as pl
from jax.experimental.pallas import tpu as pltpu, tpu_sc as plsc

N_NODES = 10000
N_EDGES = 320000
D = 128
NP = 2
DH = D // NP
B = 2
NS = 16
VL = 16
EPT = N_EDGES // NS
CH = 80
NCH = EPT // CH
NPAIR = NCH // 2
RPT = 624
WBC = 80


def _sc_body(n1_hbm, n2_hbm, w_hbm, h_hbm, out_lo, out_hi,
             ids_v, n2_v, w_v, tab_v, idxz_v,
             rowsa_v, rowsb_v, scta_v, sctb_v,
             idx1a_v, idx2a_v, idx1b_v, idx2b_v,
             acc_s, cnt_s, gsema, gsemb, ssema, ssemb, asem):
    c = lax.axis_index("c")
    t = lax.axis_index("s")
    zeros_i = jnp.zeros((VL,), jnp.int32)
    ones_i = jnp.ones((VL,), jnp.int32)
    zeros_f = jnp.zeros((VL,), jnp.float32)

    def _zt(i, _):
        tab_v[i] = zeros_i
        return 0
    lax.fori_loop(0, N_NODES // VL, _zt, 0)

    @pl.when(t == 0)
    def _():
        pltpu.sync_copy(tab_v, cnt_s)

    for j in range(5):
        for g in range(8):
            off = min(g * VL, 125 - VL)
            idxz_v[j, pl.ds(off, VL)] = (
                lax.iota(jnp.int32, VL) + (j * 125 + off))

    plsc.subcore_barrier()

    cols = [(n1_hbm, 0, ids_v), (n1_hbm, 1, n2_v),
            (n2_hbm, 0, ids_v), (n2_hbm, 1, n2_v)]
    ref0, b0, buf0 = cols[0]
    pltpu.async_copy(ref0.at[pl.ds(b0 * N_EDGES + t * EPT, EPT)], buf0, asem)
    for ci, (ref, b, buf) in enumerate(cols):
        pltpu.make_async_copy(
            ref.at[pl.ds(b * N_EDGES + t * EPT, EPT)], buf, asem).wait()
        if ci + 1 < len(cols):
            refn, bn, bufn = cols[ci + 1]
            pltpu.async_copy(
                refn.at[pl.ds(bn * N_EDGES + t * EPT, EPT)], bufn, asem)

        def _mark(i, _):
            v = buf[pl.ds(i * VL, VL)]
            row = lax.shift_right_logical(v, 4)
            col = lax.bitwise_and(v, jnp.int32(15))
            plsc.store_scatter(tab_v, [row, col], ones_i)
            return 0
        lax.fori_loop(0, EPT // VL, _mark, 0, unroll=5)

    for j in range(5):
        pltpu.sync_copy(tab_v.at[pl.ds(j * 125, 125)],
                        cnt_s.at[idxz_v.at[j]], add=True)
    plsc.subcore_barrier()
    pltpu.sync_copy(cnt_s, tab_v)

    def _rank(i, carry):
        p = (tab_v[i] > 0).astype(jnp.int32)
        inc = plsc.cumsum(p)
        tab_v[i] = carry + inc - p
        return carry + jnp.sum(p)
    lax.fori_loop(0, N_NODES // VL, _rank, jnp.int32(0))

    ebase = c * N_EDGES + t * EPT
    pltpu.sync_copy(n1_hbm.at[pl.ds(ebase, EPT)], ids_v)
    pltpu.sync_copy(n2_hbm.at[pl.ds(ebase, EPT)], n2_v)
    pltpu.sync_copy(w_hbm.at[pl.ds(ebase, EPT)], w_v)
    hoff = c * N_NODES

    def _ranks0(base, idx1_ref, idx2_ref):
        for g in range(CH // VL):
            o = base + g * VL
            v1 = ids_v[pl.ds(o, VL)]
            v2 = n2_v[pl.ds(o, VL)]
            fifteen = jnp.int32(15)
            r1 = plsc.load_gather(
                tab_v, [lax.shift_right_logical(v1, 4),
                        lax.bitwise_and(v1, fifteen)])
            r2 = plsc.load_gather(
                tab_v, [lax.shift_right_logical(v2, 4),
                        lax.bitwise_and(v2, fifteen)])
            r2 = (r2 + hoff) * NP
            ids_v[pl.ds(o, VL)] = r1
            n2_v[pl.ds(o, VL)] = r2
            idx1_ref[pl.ds(g * VL, VL)] = r1
            idx2_ref[pl.ds(g * VL, VL)] = r2

    def _ranksn(base, idx1_ref, idx2_ref, d):
        for g in range(CH // VL):
            o = base + g * VL
            idx1_ref[pl.ds(g * VL, VL)] = ids_v[pl.ds(o, VL)]
            idx2_ref[pl.ds(g * VL, VL)] = n2_v[pl.ds(o, VL)] + d

    def _ranks(i, idx1_ref, idx2_ref, d):
        if d == 0:
            _ranks0(i * CH, idx1_ref, idx2_ref)
        else:
            _ranksn(i * CH, idx1_ref, idx2_ref, d)

    def _scale(base, rows_ref, sct_ref):
        for g in range(CH // VL):
            wv = w_v[pl.ds(base + g * VL, VL)]
            for e in range(VL):
                ws = wv[e]
                r = g * VL + e
                for half in range(DH // 32):
                    x = rows_ref[r, pl.ds(half * 32, 32)]
                    a, b = plsc.unpack(
                        x, format=plsc.PackFormat.INTERLEAVED,
                        preferred_element_type=jnp.float32)
                    sct_ref[r, pl.ds(half * 32, VL)] = a * ws
                    sct_ref[r, pl.ds(half * 32 + VL, VL)] = b * ws

    for d, out_ref in ((0, out_lo), (1, out_hi)):
        def _zr(r, _):
            for dd in range(DH // VL):
                scta_v[r, pl.ds(dd * VL, VL)] = zeros_f
            return 0
        lax.fori_loop(0, WBC, _zr, 0)
        for k in range(8):
            pltpu.sync_copy(scta_v, acc_s.at[pl.ds(t * RPT + k * WBC, WBC)])
        plsc.subcore_barrier()

        _ranks(0, idx1a_v, idx2a_v, d)
        pltpu.async_copy(h_hbm.at[idx2a_v], rowsa_v, gsema)

        def _pair(j, _):
            a = 2 * j
            b = a + 1

            @pl.when(j > 0)
            def _():
                pltpu.make_async_copy(sctb_v, acc_s.at[idx1b_v],
                                      ssemb).wait()
            _ranks(b, idx1b_v, idx2b_v, d)
            pltpu.async_copy(h_hbm.at[idx2b_v], rowsb_v, gsemb)

            pltpu.make_async_copy(h_hbm.at[idx2a_v], rowsa_v, gsema).wait()
            _scale(a * CH, rowsa_v, scta_v)
            pltpu.async_copy(scta_v, acc_s.at[idx1a_v], ssema, add=True)

            @pl.when(j < NPAIR - 1)
            def _():
                pltpu.make_async_copy(scta_v, acc_s.at[idx1a_v],
                                      ssema).wait()
                _ranks(a + 2, idx1a_v, idx2a_v, d)
                pltpu.async_copy(h_hbm.at[idx2a_v], rowsa_v, gsema)

            pltpu.make_async_copy(h_hbm.at[idx2b_v], rowsb_v, gsemb).wait()
            _scale(b * CH, rowsb_v, sctb_v)
            pltpu.async_copy(sctb_v, acc_s.at[idx1b_v], ssemb, add=True)
            return 0
        lax.fori_loop(0, NPAIR, _pair, 0)
        pltpu.make_async_copy(scta_v, acc_s.at[idx1a_v], ssema).wait()
        pltpu.make_async_copy(sctb_v, acc_s.at[idx1b_v], ssemb).wait()

        plsc.subcore_barrier()
        for k in range(8):
            pltpu.sync_copy(acc_s.at[pl.ds(t * RPT + k * WBC, WBC)],
                            scta_v)
            rbase = pl.multiple_of(c * N_NODES + t * RPT + k * WBC, 8)
            pltpu.sync_copy(scta_v, out_ref.at[pl.ds(rbase, WBC)])
        plsc.subcore_barrier()


_mesh = plsc.VectorSubcoreMesh(core_axis_name="c", subcore_axis_name="s")

_sc_call = pl.kernel(
    _sc_body,
    out_type=(
        jax.ShapeDtypeStruct((B * N_NODES, DH), jnp.float32),
        jax.ShapeDtypeStruct((B * N_NODES, DH), jnp.float32),
    ),
    mesh=_mesh,
    compiler_params=pltpu.CompilerParams(
        needs_layout_passes=False, use_tc_tiling_on_sc=False),
    scratch_types=[
        pltpu.VMEM((EPT,), jnp.int32),
        pltpu.VMEM((EPT,), jnp.int32),
        pltpu.VMEM((EPT,), jnp.float32),
        pltpu.VMEM((N_NODES // VL, VL), jnp.int32),
        pltpu.VMEM((5, 125), jnp.int32),
        pltpu.VMEM((CH, DH), jnp.bfloat16),
        pltpu.VMEM((CH, DH), jnp.bfloat16),
        pltpu.VMEM((CH, DH), jnp.float32),
        pltpu.VMEM((CH, DH), jnp.float32),
        pltpu.VMEM((CH,), jnp.int32),
        pltpu.VMEM((CH,), jnp.int32),
        pltpu.VMEM((CH,), jnp.int32),
        pltpu.VMEM((CH,), jnp.int32),
        pltpu.VMEM_SHARED((N_NODES, DH), jnp.float32),
        pltpu.VMEM_SHARED((N_NODES // VL, VL), jnp.int32),
        pltpu.SemaphoreType.DMA,
        pltpu.SemaphoreType.DMA,
        pltpu.SemaphoreType.DMA,
        pltpu.SemaphoreType.DMA,
        pltpu.SemaphoreType.DMA,
    ],
)


@jax.jit
def _impl(H, edge_weights):
    n1 = edge_weights[:, :, 0].astype(jnp.int32).reshape(B * N_EDGES)
    n2 = edge_weights[:, :, 1].astype(jnp.int32).reshape(B * N_EDGES)
    w = edge_weights[:, :, 2].astype(jnp.float32).reshape(B * N_EDGES)
    hf = H.astype(jnp.bfloat16).reshape(B * N_NODES * NP, DH)
    lo, hi = _sc_call(n1, n2, w, hf)

    def _unperm(x):
        x = x.reshape(B, N_NODES, DH // 32, 2, 16)
        x = x.transpose(0, 1, 2, 4, 3)
        return x.reshape(B, N_NODES, DH)

    out = jnp.concatenate([_unperm(lo), _unperm(hi)], axis=-1)
    return out


def kernel(H, edge_weights):
    return _impl(H, edge_weights)


# speedup vs baseline: 1.0673x; 1.0673x over previous
import jax
import jax.numpy as jnp
from jax import lax
from jax.experimental import pallas as pl
from jax.experimental.pallas import tpu as pltpu, tpu_sc as plsc

N_NODES = 10000
N_EDGES = 320000
D = 128
NP = 2
DH = D // NP
B = 2
NS = 16
VL = 16
EPT = N_EDGES // NS
CH = 80
NCH = EPT // CH
NPAIR = NCH // 2
RPT = 624
WBC = 80


def _sc_body(n1_hbm, n2_hbm, w_hbm, h_hbm, out_lo, out_hi,
             ids_v, n2_v, w_v, tab_v, idxz_v, zrow_v,
             rowsa_v, rowsb_v, idx1a_v, idx2a_v, idx1b_v, idx2b_v,
             acc_s, cnt_s, gsema, gsemb, ssema, ssemb, asem):
    c = lax.axis_index("c")
    t = lax.axis_index("s")
    zeros_i = jnp.zeros((VL,), jnp.int32)
    ones_i = jnp.ones((VL,), jnp.int32)
    zeros_f = jnp.zeros((VL,), jnp.float32)

    def _zt(i, _):
        tab_v[i] = zeros_i
        return 0
    lax.fori_loop(0, N_NODES // VL, _zt, 0)

    @pl.when(t == 0)
    def _():
        pltpu.sync_copy(tab_v, cnt_s)

    for j in range(5):
        for g in range(8):
            off = min(g * VL, 125 - VL)
            idxz_v[j, pl.ds(off, VL)] = (
                lax.iota(jnp.int32, VL) + (j * 125 + off))

    def _zr(r, _):
        for dd in range(DH // VL):
            zrow_v[r, pl.ds(dd * VL, VL)] = zeros_f
        return 0
    lax.fori_loop(0, WBC, _zr, 0)

    plsc.subcore_barrier()

    cols = [(n1_hbm, 0, ids_v), (n1_hbm, 1, n2_v),
            (n2_hbm, 0, ids_v), (n2_hbm, 1, n2_v)]
    ref0, b0, buf0 = cols[0]
    pltpu.async_copy(ref0.at[pl.ds(b0 * N_EDGES + t * EPT, EPT)], buf0, asem)
    for ci, (ref, b, buf) in enumerate(cols):
        pltpu.make_async_copy(
            ref.at[pl.ds(b * N_EDGES + t * EPT, EPT)], buf, asem).wait()
        if ci + 1 < len(cols):
            refn, bn, bufn = cols[ci + 1]
            pltpu.async_copy(
                refn.at[pl.ds(bn * N_EDGES + t * EPT, EPT)], bufn, asem)

        def _mark(i, _):
            v = buf[pl.ds(i * VL, VL)]
            row = lax.shift_right_logical(v, 4)
            col = lax.bitwise_and(v, jnp.int32(15))
            plsc.store_scatter(tab_v, [row, col], ones_i)
            return 0
        lax.fori_loop(0, EPT // VL, _mark, 0, unroll=5)

    for j in range(5):
        pltpu.sync_copy(tab_v.at[pl.ds(j * 125, 125)],
                        cnt_s.at[idxz_v.at[j]], add=True)
    plsc.subcore_barrier()
    pltpu.sync_copy(cnt_s, tab_v)

    def _rank(i, carry):
        p = (tab_v[i] > 0).astype(jnp.int32)
        inc = plsc.cumsum(p)
        tab_v[i] = carry + inc - p
        return carry + jnp.sum(p)
    lax.fori_loop(0, N_NODES // VL, _rank, jnp.int32(0))

    ebase = c * N_EDGES + t * EPT
    pltpu.sync_copy(n1_hbm.at[pl.ds(ebase, EPT)], ids_v)
    pltpu.sync_copy(n2_hbm.at[pl.ds(ebase, EPT)], n2_v)
    pltpu.sync_copy(w_hbm.at[pl.ds(ebase, EPT)], w_v)
    hoff = c * N_NODES

    def _ranks0(base, idx1_ref, idx2_ref):
        for g in range(CH // VL):
            o = base + g * VL
            v1 = ids_v[pl.ds(o, VL)]
            v2 = n2_v[pl.ds(o, VL)]
            fifteen = jnp.int32(15)
            r1 = plsc.load_gather(
                tab_v, [lax.shift_right_logical(v1, 4),
                        lax.bitwise_and(v1, fifteen)])
            r2 = plsc.load_gather(
                tab_v, [lax.shift_right_logical(v2, 4),
                        lax.bitwise_and(v2, fifteen)])
            r2 = (r2 + hoff) * NP
            ids_v[pl.ds(o, VL)] = r1
            n2_v[pl.ds(o, VL)] = r2
            idx1_ref[pl.ds(g * VL, VL)] = r1
            idx2_ref[pl.ds(g * VL, VL)] = r2

    def _ranksn(base, idx1_ref, idx2_ref, d):
        for g in range(CH // VL):
            o = base + g * VL
            idx1_ref[pl.ds(g * VL, VL)] = ids_v[pl.ds(o, VL)]
            idx2_ref[pl.ds(g * VL, VL)] = n2_v[pl.ds(o, VL)] + d

    def _ranks(i, idx1_ref, idx2_ref, d):
        if d == 0:
            _ranks0(i * CH, idx1_ref, idx2_ref)
        else:
            _ranksn(i * CH, idx1_ref, idx2_ref, d)

    def _scale(base, rows_ref):
        for g in range(CH // VL):
            wv = w_v[pl.ds(base + g * VL, VL)]
            for e in range(VL):
                ws = wv[e]
                r = g * VL + e
                for dd in range(DH // VL):
                    s = pl.ds(dd * VL, VL)
                    rows_ref[r, s] = rows_ref[r, s] * ws

    for d, out_ref in ((0, out_lo), (1, out_hi)):
        for k in range(8):
            pltpu.sync_copy(zrow_v, acc_s.at[pl.ds(t * RPT + k * WBC, WBC)])
        plsc.subcore_barrier()

        _ranks(0, idx1a_v, idx2a_v, d)
        pltpu.async_copy(h_hbm.at[idx2a_v], rowsa_v, gsema)

        def _pair(j, _):
            a = 2 * j
            b = a + 1

            @pl.when(j > 0)
            def _():
                pltpu.make_async_copy(rowsb_v, acc_s.at[idx1b_v],
                                      ssemb).wait()
            _ranks(b, idx1b_v, idx2b_v, d)
            pltpu.async_copy(h_hbm.at[idx2b_v], rowsb_v, gsemb)

            pltpu.make_async_copy(h_hbm.at[idx2a_v], rowsa_v, gsema).wait()
            _scale(a * CH, rowsa_v)
            pltpu.async_copy(rowsa_v, acc_s.at[idx1a_v], ssema, add=True)

            pltpu.make_async_copy(h_hbm.at[idx2b_v], rowsb_v, gsemb).wait()

            @pl.when(j < NPAIR - 1)
            def _():
                pltpu.make_async_copy(rowsa_v, acc_s.at[idx1a_v],
                                      ssema).wait()
                _ranks(a + 2, idx1a_v, idx2a_v, d)
                pltpu.async_copy(h_hbm.at[idx2a_v], rowsa_v, gsema)

            _scale(b * CH, rowsb_v)
            pltpu.async_copy(rowsb_v, acc_s.at[idx1b_v], ssemb, add=True)
            return 0
        lax.fori_loop(0, NPAIR, _pair, 0)
        pltpu.make_async_copy(rowsa_v, acc_s.at[idx1a_v], ssema).wait()
        pltpu.make_async_copy(rowsb_v, acc_s.at[idx1b_v], ssemb).wait()

        plsc.subcore_barrier()
        for k in range(8):
            pltpu.sync_copy(acc_s.at[pl.ds(t * RPT + k * WBC, WBC)],
                            rowsa_v)
            rbase = pl.multiple_of(c * N_NODES + t * RPT + k * WBC, 8)
            pltpu.sync_copy(rowsa_v, out_ref.at[pl.ds(rbase, WBC)])
        plsc.subcore_barrier()


_mesh = plsc.VectorSubcoreMesh(core_axis_name="c", subcore_axis_name="s")

_sc_call = pl.kernel(
    _sc_body,
    out_type=(
        jax.ShapeDtypeStruct((B * N_NODES, DH), jnp.float32),
        jax.ShapeDtypeStruct((B * N_NODES, DH), jnp.float32),
    ),
    mesh=_mesh,
    compiler_params=pltpu.CompilerParams(
        needs_layout_passes=False, use_tc_tiling_on_sc=False),
    scratch_types=[
        pltpu.VMEM((EPT,), jnp.int32),
        pltpu.VMEM((EPT,), jnp.int32),
        pltpu.VMEM((EPT,), jnp.float32),
        pltpu.VMEM((N_NODES // VL, VL), jnp.int32),
        pltpu.VMEM((5, 125), jnp.int32),
        pltpu.VMEM((WBC, DH), jnp.float32),
        pltpu.VMEM((CH, DH), jnp.float32),
        pltpu.VMEM((CH, DH), jnp.float32),
        pltpu.VMEM((CH,), jnp.int32),
        pltpu.VMEM((CH,), jnp.int32),
        pltpu.VMEM((CH,), jnp.int32),
        pltpu.VMEM((CH,), jnp.int32),
        pltpu.VMEM_SHARED((N_NODES, DH), jnp.float32),
        pltpu.VMEM_SHARED((N_NODES // VL, VL), jnp.int32),
        pltpu.SemaphoreType.DMA,
        pltpu.SemaphoreType.DMA,
        pltpu.SemaphoreType.DMA,
        pltpu.SemaphoreType.DMA,
        pltpu.SemaphoreType.DMA,
    ],
)


@jax.jit
def _impl(H, edge_weights):
    n1 = edge_weights[:, :, 0].astype(jnp.int32).reshape(B * N_EDGES)
    n2 = edge_weights[:, :, 1].astype(jnp.int32).reshape(B * N_EDGES)
    w = edge_weights[:, :, 2].astype(jnp.float32).reshape(B * N_EDGES)
    hf = H.astype(jnp.float32).reshape(B * N_NODES * NP, DH)
    lo, hi = _sc_call(n1, n2, w, hf)
    out = jnp.concatenate(
        [lo.reshape(B, N_NODES, DH), hi.reshape(B, N_NODES, DH)], axis=-1)
    return out


def kernel(H, edge_weights):
    return _impl(H, edge_weights)


# speedup vs baseline: 1.0983x; 1.0290x over previous
import jax
import jax.numpy as jnp
from jax import lax
from jax.experimental import pallas as pl
from jax.experimental.pallas import tpu as pltpu, tpu_sc as plsc

N_NODES = 10000
N_EDGES = 320000
D = 128
NP = 2
DH = D // NP
B = 2
NS = 16
VL = 16
EPT = N_EDGES // NS
CH = 80
NCH = EPT // CH
NPAIR = NCH // 2
RPT = 624
WBC = 80


def _sc_body(n1_hbm, n2_hbm, w_hbm, h_hbm, out_lo, out_hi,
             ids_v, n2_v, w_v, tab_v, idxz_v, zrow_v,
             rowsa_v, rowsb_v, idx1a_v, idx2a_v, idx1b_v, idx2b_v,
             acc_s, cnt_s, gsema, gsemb, ssema, ssemb, asem):
    c = lax.axis_index("c")
    t = lax.axis_index("s")
    zeros_i = jnp.zeros((VL,), jnp.int32)
    ones_i = jnp.ones((VL,), jnp.int32)
    zeros_f = jnp.zeros((VL,), jnp.float32)

    def _zt(i, _):
        tab_v[i] = zeros_i
        return 0
    lax.fori_loop(0, N_NODES // VL, _zt, 0)

    @pl.when(t == 0)
    def _():
        pltpu.sync_copy(tab_v, cnt_s)

    for j in range(5):
        for g in range(8):
            off = min(g * VL, 125 - VL)
            idxz_v[j, pl.ds(off, VL)] = (
                lax.iota(jnp.int32, VL) + (j * 125 + off))

    def _zr(r, _):
        for dd in range(DH // VL):
            zrow_v[r, pl.ds(dd * VL, VL)] = zeros_f
        return 0
    lax.fori_loop(0, WBC, _zr, 0)

    plsc.subcore_barrier()

    cols = [(n1_hbm, 0, ids_v), (n1_hbm, 1, n2_v),
            (n2_hbm, 0, ids_v), (n2_hbm, 1, n2_v)]
    ref0, b0, buf0 = cols[0]
    pltpu.async_copy(ref0.at[pl.ds(b0 * N_EDGES + t * EPT, EPT)], buf0, asem)
    for ci, (ref, b, buf) in enumerate(cols):
        pltpu.make_async_copy(
            ref.at[pl.ds(b * N_EDGES + t * EPT, EPT)], buf, asem).wait()
        if ci + 1 < len(cols):
            refn, bn, bufn = cols[ci + 1]
            pltpu.async_copy(
                refn.at[pl.ds(bn * N_EDGES + t * EPT, EPT)], bufn, asem)

        def _mark(i, _):
            v = buf[pl.ds(i * VL, VL)]
            row = lax.shift_right_logical(v, 4)
            col = lax.bitwise_and(v, jnp.int32(15))
            plsc.store_scatter(tab_v, [row, col], ones_i)
            return 0
        lax.fori_loop(0, EPT // VL, _mark, 0, unroll=5)

    for j in range(5):
        pltpu.sync_copy(tab_v.at[pl.ds(j * 125, 125)],
                        cnt_s.at[idxz_v.at[j]], add=True)
    plsc.subcore_barrier()
    pltpu.sync_copy(cnt_s, tab_v)

    def _rank(i, carry):
        p = (tab_v[i] > 0).astype(jnp.int32)
        inc = plsc.cumsum(p)
        tab_v[i] = carry + inc - p
        return carry + jnp.sum(p)
    lax.fori_loop(0, N_NODES // VL, _rank, jnp.int32(0))

    ebase = c * N_EDGES + t * EPT
    pltpu.sync_copy(n1_hbm.at[pl.ds(ebase, EPT)], ids_v)
    pltpu.sync_copy(n2_hbm.at[pl.ds(ebase, EPT)], n2_v)
    pltpu.sync_copy(w_hbm.at[pl.ds(ebase, EPT)], w_v)
    hoff = c * N_NODES

    def _ranks0(base, idx1_ref, idx2_ref):
        for g in range(CH // VL):
            o = base + g * VL
            v1 = ids_v[pl.ds(o, VL)]
            v2 = n2_v[pl.ds(o, VL)]
            fifteen = jnp.int32(15)
            r1 = plsc.load_gather(
                tab_v, [lax.shift_right_logical(v1, 4),
                        lax.bitwise_and(v1, fifteen)])
            r2 = plsc.load_gather(
                tab_v, [lax.shift_right_logical(v2, 4),
                        lax.bitwise_and(v2, fifteen)])
            r2 = (r2 + hoff) * NP
            ids_v[pl.ds(o, VL)] = r1
            n2_v[pl.ds(o, VL)] = r2
            idx1_ref[pl.ds(g * VL, VL)] = r1
            idx2_ref[pl.ds(g * VL, VL)] = r2

    def _ranksn(base, idx1_ref, idx2_ref, d):
        for g in range(CH // VL):
            o = base + g * VL
            idx1_ref[pl.ds(g * VL, VL)] = ids_v[pl.ds(o, VL)]
            idx2_ref[pl.ds(g * VL, VL)] = n2_v[pl.ds(o, VL)] + d

    def _ranks(i, idx1_ref, idx2_ref, d):
        if d == 0:
            _ranks0(i * CH, idx1_ref, idx2_ref)
        else:
            _ranksn(i * CH, idx1_ref, idx2_ref, d)

    def _scale(base, rows_ref):
        for g in range(CH // VL):
            wv = w_v[pl.ds(base + g * VL, VL)]
            for e in range(VL):
                ws = wv[e]
                r = g * VL + e
                for dd in range(DH // VL):
                    s = pl.ds(dd * VL, VL)
                    rows_ref[r, s] = rows_ref[r, s] * ws

    for d, out_ref in ((0, out_lo), (1, out_hi)):
        for k in range(8):
            pltpu.sync_copy(zrow_v, acc_s.at[pl.ds(t * RPT + k * WBC, WBC)])
        plsc.subcore_barrier()

        _ranks(0, idx1a_v, idx2a_v, d)
        pltpu.async_copy(h_hbm.at[idx2a_v], rowsa_v, gsema)

        def _pair(j, _):
            a = 2 * j
            b = a + 1

            @pl.when(j > 0)
            def _():
                pltpu.make_async_copy(rowsb_v, acc_s.at[idx1b_v],
                                      ssemb).wait()
            _ranks(b, idx1b_v, idx2b_v, d)
            pltpu.async_copy(h_hbm.at[idx2b_v], rowsb_v, gsemb)

            pltpu.make_async_copy(h_hbm.at[idx2a_v], rowsa_v, gsema).wait()
            _scale(a * CH, rowsa_v)
            pltpu.async_copy(rowsa_v, acc_s.at[idx1a_v], ssema, add=True)

            @pl.when(j < NPAIR - 1)
            def _():
                pltpu.make_async_copy(rowsa_v, acc_s.at[idx1a_v],
                                      ssema).wait()
                _ranks(a + 2, idx1a_v, idx2a_v, d)
                pltpu.async_copy(h_hbm.at[idx2a_v], rowsa_v, gsema)

            pltpu.make_async_copy(h_hbm.at[idx2b_v], rowsb_v, gsemb).wait()
            _scale(b * CH, rowsb_v)
            pltpu.async_copy(rowsb_v, acc_s.at[idx1b_v], ssemb, add=True)
            return 0
        lax.fori_loop(0, NPAIR, _pair, 0)
        pltpu.make_async_copy(rowsa_v, acc_s.at[idx1a_v], ssema).wait()
        pltpu.make_async_copy(rowsb_v, acc_s.at[idx1b_v], ssemb).wait()

        plsc.subcore_barrier()
        for k in range(8):
            pltpu.sync_copy(acc_s.at[pl.ds(t * RPT + k * WBC, WBC)],
                            rowsa_v)
            rbase = pl.multiple_of(c * N_NODES + t * RPT + k * WBC, 8)
            pltpu.sync_copy(rowsa_v, out_ref.at[pl.ds(rbase, WBC)])
        plsc.subcore_barrier()


_mesh = plsc.VectorSubcoreMesh(core_axis_name="c", subcore_axis_name="s")

_sc_call = pl.kernel(
    _sc_body,
    out_type=(
        jax.ShapeDtypeStruct((B * N_NODES, DH), jnp.float32),
        jax.ShapeDtypeStruct((B * N_NODES, DH), jnp.float32),
    ),
    mesh=_mesh,
    compiler_params=pltpu.CompilerParams(
        needs_layout_passes=False, use_tc_tiling_on_sc=False),
    scratch_types=[
        pltpu.VMEM((EPT,), jnp.int32),
        pltpu.VMEM((EPT,), jnp.int32),
        pltpu.VMEM((EPT,), jnp.float32),
        pltpu.VMEM((N_NODES // VL, VL), jnp.int32),
        pltpu.VMEM((5, 125), jnp.int32),
        pltpu.VMEM((WBC, DH), jnp.float32),
        pltpu.VMEM((CH, DH), jnp.float32),
        pltpu.VMEM((CH, DH), jnp.float32),
        pltpu.VMEM((CH,), jnp.int32),
        pltpu.VMEM((CH,), jnp.int32),
        pltpu.VMEM((CH,), jnp.int32),
        pltpu.VMEM((CH,), jnp.int32),
        pltpu.VMEM_SHARED((N_NODES, DH), jnp.float32),
        pltpu.VMEM_SHARED((N_NODES // VL, VL), jnp.int32),
        pltpu.SemaphoreType.DMA,
        pltpu.SemaphoreType.DMA,
        pltpu.SemaphoreType.DMA,
        pltpu.SemaphoreType.DMA,
        pltpu.SemaphoreType.DMA,
    ],
)


@jax.jit
def _impl(H, edge_weights):
    n1 = edge_weights[:, :, 0].astype(jnp.int32).reshape(B * N_EDGES)
    n2 = edge_weights[:, :, 1].astype(jnp.int32).reshape(B * N_EDGES)
    w = edge_weights[:, :, 2].astype(jnp.float32).reshape(B * N_EDGES)
    hf = H.astype(jnp.float32).reshape(B * N_NODES * NP, DH)
    lo, hi = _sc_call(n1, n2, w, hf)
    out = jnp.concatenate(
        [lo.reshape(B, N_NODES, DH), hi.reshape(B, N_NODES, DH)], axis=-1)
    return out


def kernel(H, edge_weights):
    return _impl(H, edge_weights)
